# Initial kernel scaffold; baseline (speedup 1.0000x reference)
import jax
import jax.numpy as jnp
from jax.experimental import pallas as pl


def kernel(x, edge_index, W1, b1, g1, beta1, W2, b2, g2, beta2):
    raise NotImplementedError("write your pallas kernel here")


# speedup vs baseline: 11.6692x; 11.6692x over previous
import functools

import jax
import jax.numpy as jnp
from jax import lax
from jax.experimental import pallas as pl
from jax.experimental.pallas import tpu as pltpu
from jax.experimental.pallas import tpu_sc as plsc

N = 10000
E = 320000
D = 128

NC = 2
NS = 16
NW = NC * NS
EPT = E // NW
EB = 80
NBLK = EPT // EB
ZT = 1000
NZ = N // ZT

R = 400
G = N // R
EPS = 1e-5


def _deg_body(dst_hbm, ones_hbm, zeros_hbm, out_hbm, ones_v, dst_v, acc_sh):
    c = lax.axis_index("c")
    s = lax.axis_index("s")
    base = pl.multiple_of((c * NS + s) * EPT, EB)
    row0 = pl.multiple_of(s * ZT, 8)

    @pl.when(s < NZ)
    def _zero():
        pltpu.sync_copy(zeros_hbm, acc_sh.at[pl.ds(row0, ZT)])

    pltpu.sync_copy(ones_hbm, ones_v)
    plsc.subcore_barrier()

    def body(i, carry):
        e0 = pl.multiple_of(base + i * EB, EB)
        pltpu.sync_copy(dst_hbm.at[pl.ds(e0, EB)], dst_v)
        pltpu.sync_copy(ones_v, acc_sh.at[dst_v], add=True)
        return carry

    lax.fori_loop(0, NBLK, body, 0)
    plsc.subcore_barrier()

    @pl.when(s < NZ)
    def _out():
        pltpu.sync_copy(acc_sh.at[pl.ds(row0, ZT)],
                        out_hbm.at[c, pl.ds(row0, ZT)])


@functools.lru_cache(maxsize=None)
def _sc_mesh():
    return plsc.VectorSubcoreMesh(core_axis_name="c", subcore_axis_name="s",
                                  num_cores=NC, num_subcores=NS)


@functools.lru_cache(maxsize=None)
def _deg_call():
    return pl.kernel(
        _deg_body,
        out_type=jax.ShapeDtypeStruct((NC, N, 16), jnp.float32),
        mesh=_sc_mesh(),
        scratch_types=[
            pltpu.VMEM((EB, 16), jnp.float32),
            pltpu.VMEM((EB,), jnp.int32),
            pltpu.VMEM_SHARED((N, 16), jnp.float32),
        ],
    )


def _scat_body(h_hbm, src_hbm, dst_hbm, zeros_hbm, out_hbm,
               src_v, dst_v, rows_v, acc_sh, sem):
    c = lax.axis_index("c")
    s = lax.axis_index("s")
    base = pl.multiple_of((c * NS + s) * EPT, EB)
    row0 = pl.multiple_of(s * ZT, 8)

    @pl.when(s < NZ)
    def _zero():
        pltpu.sync_copy(zeros_hbm, acc_sh.at[pl.ds(row0, ZT)])

    plsc.subcore_barrier()

    def body(i, carry):
        e0 = pl.multiple_of(base + i * EB, EB)
        pltpu.sync_copy(src_hbm.at[pl.ds(e0, EB)], src_v)
        pltpu.async_copy(h_hbm.at[src_v], rows_v, sem).wait()
        pltpu.sync_copy(dst_hbm.at[pl.ds(e0, EB)], dst_v)
        pltpu.sync_copy(rows_v, acc_sh.at[dst_v], add=True)
        return carry

    lax.fori_loop(0, NBLK, body, 0)
    plsc.subcore_barrier()

    @pl.when(s < NZ)
    def _out():
        pltpu.sync_copy(acc_sh.at[pl.ds(row0, ZT)],
                        out_hbm.at[c, pl.ds(row0, ZT)])


@functools.lru_cache(maxsize=None)
def _scat_call():
    return pl.kernel(
        _scat_body,
        out_type=jax.ShapeDtypeStruct((NC, N, D), jnp.float32),
        mesh=_sc_mesh(),
        scratch_types=[
            pltpu.VMEM((EB,), jnp.int32),
            pltpu.VMEM((EB,), jnp.int32),
            pltpu.VMEM((EB, D), jnp.float32),
            pltpu.VMEM_SHARED((N, D), jnp.float32),
            pltpu.SemaphoreType.DMA,
        ],
    )


def _dinv_block(d0, d1):
    deg = d0[:, :1] + d1[:, :1] + 1.0
    return lax.rsqrt(deg)


def _mm_scale_body(x_ref, w_ref, d0_ref, d1_ref, o_ref):
    dinv = _dinv_block(d0_ref[...], d1_ref[...])
    o_ref[...] = jnp.dot(x_ref[...], w_ref[...],
                         preferred_element_type=jnp.float32) * dinv


def _agg_stats_body(p0_ref, p1_ref, h2_ref, d0_ref, d1_ref, b_ref,
                    t_ref, st_ref):
    dinv = _dinv_block(d0_ref[...], d1_ref[...])
    t = dinv * (p0_ref[...] + p1_ref[...] + h2_ref[...]) + b_ref[0:1, :]
    t_ref[...] = t

    @pl.when(pl.program_id(0) == 0)
    def _():
        st_ref[...] = jnp.zeros_like(st_ref)

    st = jnp.concatenate(
        [jnp.sum(t, axis=0, keepdims=True),
         jnp.sum(t * t, axis=0, keepdims=True),
         jnp.zeros((6, D), jnp.float32)], axis=0)
    st_ref[...] += st


def _bn_mm_scale_body(t_ref, st_ref, g_ref, beta_ref, w_ref, d0_ref, d1_ref,
                      o_ref):
    st = st_ref[...]
    mean = st[0:1, :] * (1.0 / N)
    var = st[1:2, :] * (1.0 / N) - mean * mean
    xn = (t_ref[...] - mean) * (g_ref[0:1, :] * lax.rsqrt(var + EPS)) + beta_ref[0:1, :]
    xn = jnp.maximum(xn, 0.0)
    dinv = _dinv_block(d0_ref[...], d1_ref[...])
    o_ref[...] = jnp.dot(xn, w_ref[...],
                         preferred_element_type=jnp.float32) * dinv


def _bn_relu_body(t_ref, st_ref, g_ref, beta_ref, o_ref):
    st = st_ref[...]
    mean = st[0:1, :] * (1.0 / N)
    var = st[1:2, :] * (1.0 / N) - mean * mean
    xn = (t_ref[...] - mean) * (g_ref[0:1, :] * lax.rsqrt(var + EPS)) + beta_ref[0:1, :]
    o_ref[...] = jnp.maximum(xn, 0.0)


def _row_spec(width):
    return pl.BlockSpec((R, width), lambda i: (i, 0))


def _full_spec(rows, cols):
    return pl.BlockSpec((rows, cols), lambda i: (0, 0))


_mm_scale = pl.pallas_call(
    _mm_scale_body,
    grid=(G,),
    in_specs=[_row_spec(D), _full_spec(D, D), _row_spec(16), _row_spec(16)],
    out_specs=_row_spec(D),
    out_shape=jax.ShapeDtypeStruct((N, D), jnp.float32),
)

_agg_stats = pl.pallas_call(
    _agg_stats_body,
    grid=(G,),
    in_specs=[_row_spec(D), _row_spec(D), _row_spec(D), _row_spec(16),
              _row_spec(16), _full_spec(8, D)],
    out_specs=[_row_spec(D), _full_spec(8, D)],
    out_shape=[jax.ShapeDtypeStruct((N, D), jnp.float32),
               jax.ShapeDtypeStruct((8, D), jnp.float32)],
)

_bn_mm_scale = pl.pallas_call(
    _bn_mm_scale_body,
    grid=(G,),
    in_specs=[_row_spec(D), _full_spec(8, D), _full_spec(8, D),
              _full_spec(8, D), _full_spec(D, D), _row_spec(16),
              _row_spec(16)],
    out_specs=_row_spec(D),
    out_shape=jax.ShapeDtypeStruct((N, D), jnp.float32),
)

_bn_relu = pl.pallas_call(
    _bn_relu_body,
    grid=(G,),
    in_specs=[_row_spec(D), _full_spec(8, D), _full_spec(8, D),
              _full_spec(8, D)],
    out_specs=_row_spec(D),
    out_shape=jax.ShapeDtypeStruct((N, D), jnp.float32),
)


def _pad8(v):
    return jnp.broadcast_to(v, (8, D))


def kernel(x, edge_index, W1, b1, g1, beta1, W2, b2, g2, beta2):
    src = edge_index[0]
    dst = edge_index[1]
    ones16 = jnp.ones((EB, 16), jnp.float32)
    zeros16 = jnp.zeros((ZT, 16), jnp.float32)
    zerosD = jnp.zeros((ZT, D), jnp.float32)

    degp = _deg_call()(dst, ones16, zeros16)
    d0, d1 = degp[0], degp[1]

    h2 = _mm_scale(x, W1, d0, d1)
    parts = _scat_call()(h2, src, dst, zerosD)
    t1, st1 = _agg_stats(parts[0], parts[1], h2, d0, d1, _pad8(b1))
    h2b = _bn_mm_scale(t1, st1, _pad8(g1), _pad8(beta1), W2, d0, d1)
    parts2 = _scat_call()(h2b, src, dst, zerosD)
    t2, st2 = _agg_stats(parts2[0], parts2[1], h2b, d0, d1, _pad8(b2))
    return _bn_relu(t2, st2, _pad8(g2), _pad8(beta2))


# speedup vs baseline: 23.8732x; 2.0458x over previous
import functools

import jax
import jax.numpy as jnp
from jax import lax
from jax.experimental import pallas as pl
from jax.experimental.pallas import tpu as pltpu
from jax.experimental.pallas import tpu_sc as plsc

N = 10000
E = 320000
D = 128

NC = 2
NS = 16
NW = NC * NS
EPT = E // NW
EB = 80
NBLK = EPT // EB
ZT = 1000
NZ = N // ZT

R = 400
G = N // R
EPS = 1e-5


def _fill_idx(big_v, j, buf_v):
    for k in range(EB // 16):
        buf_v[pl.ds(k * 16, 16)] = big_v[pl.ds(j * EB + k * 16, 16)]


def _deg_body(dst_hbm, ones_hbm, zeros_hbm, out_hbm, ones_v, dsts_v, dbuf_v,
              stage_v, acc_sh):
    c = lax.axis_index("c")
    s = lax.axis_index("s")
    w = c * NS + s
    row0 = pl.multiple_of(s * ZT, 8)

    @pl.when(s < NZ)
    def _zero():
        pltpu.sync_copy(zeros_hbm, stage_v)
        pltpu.sync_copy(stage_v, acc_sh.at[pl.ds(row0, ZT)])

    pltpu.sync_copy(ones_hbm, ones_v)
    base = pl.multiple_of(w * EPT, EB)
    pltpu.sync_copy(dst_hbm.at[pl.ds(base, EPT)], dsts_v)
    plsc.subcore_barrier()

    @pl.loop(0, NBLK)
    def _blk(i):
        _fill_idx(dsts_v, i, dbuf_v)
        pltpu.sync_copy(ones_v, acc_sh.at[dbuf_v], add=True)

    plsc.subcore_barrier()

    @pl.when(s < NZ)
    def _out():
        pltpu.sync_copy(acc_sh.at[pl.ds(row0, ZT)], stage_v)
        pltpu.sync_copy(stage_v,
                        out_hbm.at[pl.ds(pl.multiple_of(c * N, 8) + row0, ZT)])


@functools.lru_cache(maxsize=None)
def _sc_mesh():
    return plsc.VectorSubcoreMesh(core_axis_name="c", subcore_axis_name="s",
                                  num_cores=NC, num_subcores=NS)


@functools.lru_cache(maxsize=None)
def _deg_call():
    return pl.kernel(
        _deg_body,
        out_type=jax.ShapeDtypeStruct((NC * N,), jnp.float32),
        mesh=_sc_mesh(),
        scratch_types=[
            pltpu.VMEM((EB,), jnp.float32),
            pltpu.VMEM((EPT,), jnp.int32),
            pltpu.VMEM((EB,), jnp.int32),
            pltpu.VMEM((ZT,), jnp.float32),
            pltpu.VMEM_SHARED((N,), jnp.float32),
        ],
    )


def _scat_body(h_hbm, src_hbm, dst_hbm, zeros_hbm, out_hbm,
               srcs_v, dsts_v, sbuf_a, sbuf_b, dbuf_a, dbuf_b,
               rows_a, rows_b, acc_sh, sem_a, sem_b):
    c = lax.axis_index("c")
    s = lax.axis_index("s")
    w = c * NS + s
    row0 = pl.multiple_of(s * ZT, 8)

    @pl.when(s < NZ)
    def _zero():
        pltpu.sync_copy(zeros_hbm, acc_sh.at[pl.ds(row0, ZT)])

    base = pl.multiple_of(w * EPT, EB)
    pltpu.sync_copy(src_hbm.at[pl.ds(base, EPT)], srcs_v)
    pltpu.sync_copy(dst_hbm.at[pl.ds(base, EPT)], dsts_v)
    plsc.subcore_barrier()

    bufs = ((sbuf_a, dbuf_a, rows_a, sem_a), (sbuf_b, dbuf_b, rows_b, sem_b))
    _fill_idx(srcs_v, 0, sbuf_a)
    pltpu.async_copy(h_hbm.at[sbuf_a], rows_a, sem_a)
    _fill_idx(srcs_v, 1, sbuf_b)
    pltpu.async_copy(h_hbm.at[sbuf_b], rows_b, sem_b)

    @pl.loop(0, NBLK, step=2)
    def _blk(i):
        for b, (sbuf, dbuf, rows, sem) in enumerate(bufs):
            j = i + b

            @pl.when(j < NBLK)
            def _do():
                pltpu.make_async_copy(h_hbm.at[sbuf], rows, sem).wait()
                _fill_idx(dsts_v, j, dbuf)
                pltpu.sync_copy(rows, acc_sh.at[dbuf], add=True)

                @pl.when(j + 2 < NBLK)
                def _next():
                    _fill_idx(srcs_v, j + 2, sbuf)
                    pltpu.async_copy(h_hbm.at[sbuf], rows, sem)

    plsc.subcore_barrier()

    @pl.when(s < NZ)
    def _out():
        pltpu.sync_copy(acc_sh.at[pl.ds(row0, ZT)],
                        out_hbm.at[c, pl.ds(row0, ZT)])


@functools.lru_cache(maxsize=None)
def _scat_call():
    return pl.kernel(
        _scat_body,
        out_type=jax.ShapeDtypeStruct((NC, N, D), jnp.float32),
        mesh=_sc_mesh(),
        scratch_types=[
            pltpu.VMEM((EPT,), jnp.int32),
            pltpu.VMEM((EPT,), jnp.int32),
            pltpu.VMEM((EB,), jnp.int32),
            pltpu.VMEM((EB,), jnp.int32),
            pltpu.VMEM((EB,), jnp.int32),
            pltpu.VMEM((EB,), jnp.int32),
            pltpu.VMEM((EB, D), jnp.float32),
            pltpu.VMEM((EB, D), jnp.float32),
            pltpu.VMEM_SHARED((N, D), jnp.float32),
            pltpu.SemaphoreType.DMA,
            pltpu.SemaphoreType.DMA,
        ],
    )


def _dinv_block(d0, d1):
    deg = d0 + d1 + 1.0
    return lax.rsqrt(deg)


def _mm_scale_body(x_ref, w_ref, d0_ref, d1_ref, o_ref):
    dinv = _dinv_block(d0_ref[...], d1_ref[...])
    o_ref[...] = jnp.dot(x_ref[...], w_ref[...],
                         preferred_element_type=jnp.float32) * dinv


def _agg_stats_body(p0_ref, p1_ref, h2_ref, d0_ref, d1_ref, b_ref,
                    t_ref, st_ref):
    dinv = _dinv_block(d0_ref[...], d1_ref[...])
    t = dinv * (p0_ref[...] + p1_ref[...] + h2_ref[...]) + b_ref[0:1, :]
    t_ref[...] = t

    @pl.when(pl.program_id(0) == 0)
    def _():
        st_ref[...] = jnp.zeros_like(st_ref)

    st = jnp.concatenate(
        [jnp.sum(t, axis=0, keepdims=True),
         jnp.sum(t * t, axis=0, keepdims=True),
         jnp.zeros((6, D), jnp.float32)], axis=0)
    st_ref[...] += st


def _bn_mm_scale_body(t_ref, st_ref, g_ref, beta_ref, w_ref, d0_ref, d1_ref,
                      o_ref):
    st = st_ref[...]
    mean = st[0:1, :] * (1.0 / N)
    var = st[1:2, :] * (1.0 / N) - mean * mean
    xn = (t_ref[...] - mean) * (g_ref[0:1, :] * lax.rsqrt(var + EPS)) + beta_ref[0:1, :]
    xn = jnp.maximum(xn, 0.0)
    dinv = _dinv_block(d0_ref[...], d1_ref[...])
    o_ref[...] = jnp.dot(xn, w_ref[...],
                         preferred_element_type=jnp.float32) * dinv


def _bn_relu_body(t_ref, st_ref, g_ref, beta_ref, o_ref):
    st = st_ref[...]
    mean = st[0:1, :] * (1.0 / N)
    var = st[1:2, :] * (1.0 / N) - mean * mean
    xn = (t_ref[...] - mean) * (g_ref[0:1, :] * lax.rsqrt(var + EPS)) + beta_ref[0:1, :]
    o_ref[...] = jnp.maximum(xn, 0.0)


def _row_spec(width):
    return pl.BlockSpec((R, width), lambda i: (i, 0))


def _full_spec(rows, cols):
    return pl.BlockSpec((rows, cols), lambda i: (0, 0))


_mm_scale = pl.pallas_call(
    _mm_scale_body,
    grid=(G,),
    in_specs=[_row_spec(D), _full_spec(D, D), _row_spec(1), _row_spec(1)],
    out_specs=_row_spec(D),
    out_shape=jax.ShapeDtypeStruct((N, D), jnp.float32),
)

_agg_stats = pl.pallas_call(
    _agg_stats_body,
    grid=(G,),
    in_specs=[_row_spec(D), _row_spec(D), _row_spec(D), _row_spec(1),
              _row_spec(1), _full_spec(8, D)],
    out_specs=[_row_spec(D), _full_spec(8, D)],
    out_shape=[jax.ShapeDtypeStruct((N, D), jnp.float32),
               jax.ShapeDtypeStruct((8, D), jnp.float32)],
)

_bn_mm_scale = pl.pallas_call(
    _bn_mm_scale_body,
    grid=(G,),
    in_specs=[_row_spec(D), _full_spec(8, D), _full_spec(8, D),
              _full_spec(8, D), _full_spec(D, D), _row_spec(1),
              _row_spec(1)],
    out_specs=_row_spec(D),
    out_shape=jax.ShapeDtypeStruct((N, D), jnp.float32),
)

_bn_relu = pl.pallas_call(
    _bn_relu_body,
    grid=(G,),
    in_specs=[_row_spec(D), _full_spec(8, D), _full_spec(8, D),
              _full_spec(8, D)],
    out_specs=_row_spec(D),
    out_shape=jax.ShapeDtypeStruct((N, D), jnp.float32),
)


def _pad8(v):
    return jnp.broadcast_to(v, (8, D))


def kernel(x, edge_index, W1, b1, g1, beta1, W2, b2, g2, beta2):
    src = edge_index[0]
    dst = edge_index[1]
    ones1 = jnp.ones((EB,), jnp.float32)
    zeros1 = jnp.zeros((ZT,), jnp.float32)
    zerosD = jnp.zeros((ZT, D), jnp.float32)

    degf = _deg_call()(dst, ones1, zeros1)
    d0 = degf[:N].reshape(N, 1)
    d1 = degf[N:].reshape(N, 1)

    h2 = _mm_scale(x, W1, d0, d1)
    parts = _scat_call()(h2, src, dst, zerosD)
    t1, st1 = _agg_stats(parts[0], parts[1], h2, d0, d1, _pad8(b1))
    h2b = _bn_mm_scale(t1, st1, _pad8(g1), _pad8(beta1), W2, d0, d1)
    parts2 = _scat_call()(h2b, src, dst, zerosD)
    t2, st2 = _agg_stats(parts2[0], parts2[1], h2b, d0, d1, _pad8(b2))
    return _bn_relu(t2, st2, _pad8(g2), _pad8(beta2))


# speedup vs baseline: 26.5020x; 1.1101x over previous
import functools

import jax
import jax.numpy as jnp
from jax import lax
from jax.experimental import pallas as pl
from jax.experimental.pallas import tpu as pltpu
from jax.experimental.pallas import tpu_sc as plsc

N = 10000
E = 320000
D = 128

NC = 2
NS = 16
NW = NC * NS
EPT = E // NW
EB = 80
NBLK = EPT // EB
ZT = 1000
NZ = N // ZT

R = 400
G = N // R
EPS = 1e-5


def _fill_idx(big_v, j, buf_v):
    for k in range(EB // 16):
        buf_v[pl.ds(k * 16, 16)] = big_v[pl.ds(j * EB + k * 16, 16)]


def _deg_body(dst_hbm, ones_hbm, zeros_hbm, out_hbm, ones_v, dsts_v, dbuf_v,
              stage_v, acc_sh):
    c = lax.axis_index("c")
    s = lax.axis_index("s")
    w = c * NS + s
    row0 = pl.multiple_of(s * ZT, 8)

    @pl.when(s < NZ)
    def _zero():
        pltpu.sync_copy(zeros_hbm, stage_v)
        pltpu.sync_copy(stage_v, acc_sh.at[pl.ds(row0, ZT)])

    pltpu.sync_copy(ones_hbm, ones_v)
    base = pl.multiple_of(w * EPT, EB)
    pltpu.sync_copy(dst_hbm.at[pl.ds(base, EPT)], dsts_v)
    plsc.subcore_barrier()

    @pl.loop(0, NBLK)
    def _blk(i):
        _fill_idx(dsts_v, i, dbuf_v)
        pltpu.sync_copy(ones_v, acc_sh.at[dbuf_v], add=True)

    plsc.subcore_barrier()

    @pl.when(s < NZ)
    def _out():
        pltpu.sync_copy(acc_sh.at[pl.ds(row0, ZT)], stage_v)
        pltpu.sync_copy(stage_v,
                        out_hbm.at[pl.ds(pl.multiple_of(c * N, 8) + row0, ZT)])


@functools.lru_cache(maxsize=None)
def _sc_mesh():
    return plsc.VectorSubcoreMesh(core_axis_name="c", subcore_axis_name="s",
                                  num_cores=NC, num_subcores=NS)


@functools.lru_cache(maxsize=None)
def _deg_call():
    return pl.kernel(
        _deg_body,
        out_type=jax.ShapeDtypeStruct((NC * N,), jnp.float32),
        mesh=_sc_mesh(),
        scratch_types=[
            pltpu.VMEM((EB,), jnp.float32),
            pltpu.VMEM((EPT,), jnp.int32),
            pltpu.VMEM((EB,), jnp.int32),
            pltpu.VMEM((ZT,), jnp.float32),
            pltpu.VMEM_SHARED((N,), jnp.float32),
        ],
    )


def _scat_body(h_hbm, src_hbm, dst_hbm, zeros_hbm, out_hbm,
               srcs_v, dbuf_0, dbuf_1, dbuf_2,
               rows_0, rows_1, rows_2, acc_sh,
               gsem_0, gsem_1, gsem_2, ssem_0, ssem_1, ssem_2):
    c = lax.axis_index("c")
    s = lax.axis_index("s")
    w = c * NS + s
    row0 = pl.multiple_of(s * ZT, 8)

    @pl.when(s < NZ)
    def _zero():
        pltpu.sync_copy(zeros_hbm, acc_sh.at[pl.ds(row0, ZT)])

    base = pl.multiple_of(w * EPT, EB)
    pltpu.sync_copy(src_hbm.at[pl.ds(base, EPT)], srcs_v)
    plsc.subcore_barrier()

    def _src_blk(j):
        return srcs_v.at[pl.ds(pl.multiple_of(j * EB, 8), EB)]

    def _dst_blk(j):
        return dst_hbm.at[pl.ds(pl.multiple_of(base + j * EB, 8), EB)]

    dbuf = (dbuf_0, dbuf_1, dbuf_2)
    rows = (rows_0, rows_1, rows_2)
    gsem = (gsem_0, gsem_1, gsem_2)
    ssem = (ssem_0, ssem_1, ssem_2)

    def _issue(j, b):
        pltpu.async_copy(_dst_blk(j), dbuf[b], gsem[b])
        pltpu.async_copy(h_hbm.at[_src_blk(j)], rows[b], gsem[b])

    def _await(j, b):
        pltpu.make_async_copy(_dst_blk(j), dbuf[b], gsem[b]).wait()
        pltpu.make_async_copy(h_hbm.at[_src_blk(j)], rows[b], gsem[b]).wait()

    _issue(0, 0)
    _issue(1, 1)

    @pl.loop(0, NBLK, step=3)
    def _blk(i):
        for b in range(3):
            j = i + b
            b2 = (b + 2) % 3

            @pl.when(j < NBLK)
            def _do():
                _await(j, b)
                pltpu.async_copy(rows[b], acc_sh.at[dbuf[b]], ssem[b],
                                 add=True)

                @pl.when(j + 2 < NBLK)
                def _next():
                    @pl.when(j >= 1)
                    def _free():
                        pltpu.make_async_copy(rows[b2], acc_sh.at[dbuf[b2]],
                                              ssem[b2]).wait()
                    _issue(j + 2, b2)

    for j in (NBLK - 3, NBLK - 2, NBLK - 1):
        b = j % 3
        pltpu.make_async_copy(rows[b], acc_sh.at[dbuf[b]], ssem[b]).wait()

    plsc.subcore_barrier()

    @pl.when(s < NZ)
    def _out():
        pltpu.sync_copy(acc_sh.at[pl.ds(row0, ZT)],
                        out_hbm.at[c, pl.ds(row0, ZT)])


@functools.lru_cache(maxsize=None)
def _scat_call():
    return pl.kernel(
        _scat_body,
        out_type=jax.ShapeDtypeStruct((NC, N, D), jnp.float32),
        mesh=_sc_mesh(),
        scratch_types=[
            pltpu.VMEM((EPT,), jnp.int32),
            pltpu.VMEM((EB,), jnp.int32),
            pltpu.VMEM((EB,), jnp.int32),
            pltpu.VMEM((EB,), jnp.int32),
            pltpu.VMEM((EB, D), jnp.float32),
            pltpu.VMEM((EB, D), jnp.float32),
            pltpu.VMEM((EB, D), jnp.float32),
            pltpu.VMEM_SHARED((N, D), jnp.float32),
            pltpu.SemaphoreType.DMA,
            pltpu.SemaphoreType.DMA,
            pltpu.SemaphoreType.DMA,
            pltpu.SemaphoreType.DMA,
            pltpu.SemaphoreType.DMA,
            pltpu.SemaphoreType.DMA,
        ],
    )


def _dinv_block(d0, d1):
    deg = d0 + d1 + 1.0
    return lax.rsqrt(deg)


def _mm_scale_body(x_ref, w_ref, d0_ref, d1_ref, o_ref):
    dinv = _dinv_block(d0_ref[...], d1_ref[...])
    o_ref[...] = jnp.dot(x_ref[...], w_ref[...],
                         preferred_element_type=jnp.float32) * dinv


def _agg_stats_body(p0_ref, p1_ref, h2_ref, d0_ref, d1_ref, b_ref,
                    t_ref, st_ref):
    dinv = _dinv_block(d0_ref[...], d1_ref[...])
    t = dinv * (p0_ref[...] + p1_ref[...] + h2_ref[...]) + b_ref[0:1, :]
    t_ref[...] = t

    @pl.when(pl.program_id(0) == 0)
    def _():
        st_ref[...] = jnp.zeros_like(st_ref)

    st = jnp.concatenate(
        [jnp.sum(t, axis=0, keepdims=True),
         jnp.sum(t * t, axis=0, keepdims=True),
         jnp.zeros((6, D), jnp.float32)], axis=0)
    st_ref[...] += st


def _bn_mm_scale_body(t_ref, st_ref, g_ref, beta_ref, w_ref, d0_ref, d1_ref,
                      o_ref):
    st = st_ref[...]
    mean = st[0:1, :] * (1.0 / N)
    var = st[1:2, :] * (1.0 / N) - mean * mean
    xn = (t_ref[...] - mean) * (g_ref[0:1, :] * lax.rsqrt(var + EPS)) + beta_ref[0:1, :]
    xn = jnp.maximum(xn, 0.0)
    dinv = _dinv_block(d0_ref[...], d1_ref[...])
    o_ref[...] = jnp.dot(xn, w_ref[...],
                         preferred_element_type=jnp.float32) * dinv


def _bn_relu_body(t_ref, st_ref, g_ref, beta_ref, o_ref):
    st = st_ref[...]
    mean = st[0:1, :] * (1.0 / N)
    var = st[1:2, :] * (1.0 / N) - mean * mean
    xn = (t_ref[...] - mean) * (g_ref[0:1, :] * lax.rsqrt(var + EPS)) + beta_ref[0:1, :]
    o_ref[...] = jnp.maximum(xn, 0.0)


def _row_spec(width):
    return pl.BlockSpec((R, width), lambda i: (i, 0))


def _full_spec(rows, cols):
    return pl.BlockSpec((rows, cols), lambda i: (0, 0))


_mm_scale = pl.pallas_call(
    _mm_scale_body,
    grid=(G,),
    in_specs=[_row_spec(D), _full_spec(D, D), _row_spec(1), _row_spec(1)],
    out_specs=_row_spec(D),
    out_shape=jax.ShapeDtypeStruct((N, D), jnp.float32),
)

_agg_stats = pl.pallas_call(
    _agg_stats_body,
    grid=(G,),
    in_specs=[_row_spec(D), _row_spec(D), _row_spec(D), _row_spec(1),
              _row_spec(1), _full_spec(8, D)],
    out_specs=[_row_spec(D), _full_spec(8, D)],
    out_shape=[jax.ShapeDtypeStruct((N, D), jnp.float32),
               jax.ShapeDtypeStruct((8, D), jnp.float32)],
)

_bn_mm_scale = pl.pallas_call(
    _bn_mm_scale_body,
    grid=(G,),
    in_specs=[_row_spec(D), _full_spec(8, D), _full_spec(8, D),
              _full_spec(8, D), _full_spec(D, D), _row_spec(1),
              _row_spec(1)],
    out_specs=_row_spec(D),
    out_shape=jax.ShapeDtypeStruct((N, D), jnp.float32),
)

_bn_relu = pl.pallas_call(
    _bn_relu_body,
    grid=(G,),
    in_specs=[_row_spec(D), _full_spec(8, D), _full_spec(8, D),
              _full_spec(8, D)],
    out_specs=_row_spec(D),
    out_shape=jax.ShapeDtypeStruct((N, D), jnp.float32),
)


def _pad8(v):
    return jnp.broadcast_to(v, (8, D))


def kernel(x, edge_index, W1, b1, g1, beta1, W2, b2, g2, beta2):
    src = edge_index[0]
    dst = edge_index[1]
    ones1 = jnp.ones((EB,), jnp.float32)
    zeros1 = jnp.zeros((ZT,), jnp.float32)
    zerosD = jnp.zeros((ZT, D), jnp.float32)

    degf = _deg_call()(dst, ones1, zeros1)
    d0 = degf[:N].reshape(N, 1)
    d1 = degf[N:].reshape(N, 1)

    h2 = _mm_scale(x, W1, d0, d1)
    parts = _scat_call()(h2, src, dst, zerosD)
    t1, st1 = _agg_stats(parts[0], parts[1], h2, d0, d1, _pad8(b1))
    h2b = _bn_mm_scale(t1, st1, _pad8(g1), _pad8(beta1), W2, d0, d1)
    parts2 = _scat_call()(h2b, src, dst, zerosD)
    t2, st2 = _agg_stats(parts2[0], parts2[1], h2b, d0, d1, _pad8(b2))
    return _bn_relu(t2, st2, _pad8(g2), _pad8(beta2))


# speedup vs baseline: 26.6284x; 1.0048x over previous
import functools

import jax
import jax.numpy as jnp
from jax import lax
from jax.experimental import pallas as pl
from jax.experimental.pallas import tpu as pltpu
from jax.experimental.pallas import tpu_sc as plsc

N = 10000
E = 320000
D = 128

NC = 2
NS = 16
NW = NC * NS
EPT = E // NW
EB = 80
NBLK = EPT // EB
ZT = 1000
NZ = N // ZT

R = 400
G = N // R
EPS = 1e-5


def _fill_idx(big_v, j, buf_v):
    for k in range(EB // 16):
        buf_v[pl.ds(k * 16, 16)] = big_v[pl.ds(j * EB + k * 16, 16)]


def _deg_body(dst_hbm, ones_hbm, zeros_hbm, out_hbm, ones_v, dsts_v, dbuf_v,
              stage_v, acc_sh):
    c = lax.axis_index("c")
    s = lax.axis_index("s")
    w = c * NS + s
    row0 = pl.multiple_of(s * ZT, 8)

    @pl.when(s < NZ)
    def _zero():
        pltpu.sync_copy(zeros_hbm, stage_v)
        pltpu.sync_copy(stage_v, acc_sh.at[pl.ds(row0, ZT)])

    pltpu.sync_copy(ones_hbm, ones_v)
    base = pl.multiple_of(w * EPT, EB)
    pltpu.sync_copy(dst_hbm.at[pl.ds(base, EPT)], dsts_v)
    plsc.subcore_barrier()

    @pl.loop(0, NBLK)
    def _blk(i):
        _fill_idx(dsts_v, i, dbuf_v)
        pltpu.sync_copy(ones_v, acc_sh.at[dbuf_v], add=True)

    plsc.subcore_barrier()

    @pl.when(s < NZ)
    def _out():
        pltpu.sync_copy(acc_sh.at[pl.ds(row0, ZT)], stage_v)
        pltpu.sync_copy(stage_v,
                        out_hbm.at[pl.ds(pl.multiple_of(c * N, 8) + row0, ZT)])


@functools.lru_cache(maxsize=None)
def _sc_mesh():
    return plsc.VectorSubcoreMesh(core_axis_name="c", subcore_axis_name="s",
                                  num_cores=NC, num_subcores=NS)


@functools.lru_cache(maxsize=None)
def _deg_call():
    return pl.kernel(
        _deg_body,
        out_type=jax.ShapeDtypeStruct((NC * N,), jnp.float32),
        mesh=_sc_mesh(),
        scratch_types=[
            pltpu.VMEM((EB,), jnp.float32),
            pltpu.VMEM((EPT,), jnp.int32),
            pltpu.VMEM((EB,), jnp.int32),
            pltpu.VMEM((ZT,), jnp.float32),
            pltpu.VMEM_SHARED((N,), jnp.float32),
        ],
    )


def _scat_body(h_hbm, src_hbm, dst_hbm, zeros_hbm, out_hbm,
               srcs_v, dbuf_0, dbuf_1, dbuf_2,
               rows_0, rows_1, rows_2, acc_sh,
               gsem_0, gsem_1, gsem_2, ssem_0, ssem_1, ssem_2):
    c = lax.axis_index("c")
    s = lax.axis_index("s")
    w = c * NS + s
    row0 = pl.multiple_of(s * ZT, 8)

    @pl.when(s < NZ)
    def _zero():
        pltpu.sync_copy(zeros_hbm, acc_sh.at[pl.ds(row0, ZT)])

    base = pl.multiple_of(w * EPT, EB)
    pltpu.sync_copy(src_hbm.at[pl.ds(base, EPT)], srcs_v)
    plsc.subcore_barrier()

    def _src_blk(j):
        return srcs_v.at[pl.ds(pl.multiple_of(j * EB, 8), EB)]

    def _dst_blk(j):
        return dst_hbm.at[pl.ds(pl.multiple_of(base + j * EB, 8), EB)]

    dbuf = (dbuf_0, dbuf_1, dbuf_2)
    rows = (rows_0, rows_1, rows_2)
    gsem = (gsem_0, gsem_1, gsem_2)
    ssem = (ssem_0, ssem_1, ssem_2)

    def _issue(j, b):
        pltpu.async_copy(_dst_blk(j), dbuf[b], gsem[b])
        pltpu.async_copy(h_hbm.at[_src_blk(j)], rows[b], gsem[b])

    def _await(j, b):
        pltpu.make_async_copy(_dst_blk(j), dbuf[b], gsem[b]).wait()
        pltpu.make_async_copy(h_hbm.at[_src_blk(j)], rows[b], gsem[b]).wait()

    _issue(0, 0)
    _issue(1, 1)

    @pl.loop(0, NBLK, step=3)
    def _blk(i):
        for b in range(3):
            j = i + b
            b2 = (b + 2) % 3

            @pl.when(j < NBLK)
            def _do():
                _await(j, b)
                pltpu.async_copy(rows[b], acc_sh.at[dbuf[b]], ssem[b],
                                 add=True)

                @pl.when(j + 2 < NBLK)
                def _next():
                    @pl.when(j >= 1)
                    def _free():
                        pltpu.make_async_copy(rows[b2], acc_sh.at[dbuf[b2]],
                                              ssem[b2]).wait()
                    _issue(j + 2, b2)

    for j in (NBLK - 3, NBLK - 2, NBLK - 1):
        b = j % 3
        pltpu.make_async_copy(rows[b], acc_sh.at[dbuf[b]], ssem[b]).wait()

    plsc.subcore_barrier()

    @pl.when(s < NZ)
    def _out():
        pltpu.sync_copy(acc_sh.at[pl.ds(row0, ZT)],
                        out_hbm.at[c, pl.ds(row0, ZT)])


@functools.lru_cache(maxsize=None)
def _scat_call():
    return pl.kernel(
        _scat_body,
        out_type=jax.ShapeDtypeStruct((NC, N, D), jnp.float32),
        mesh=_sc_mesh(),
        scratch_types=[
            pltpu.VMEM((EPT,), jnp.int32),
            pltpu.VMEM((EB,), jnp.int32),
            pltpu.VMEM((EB,), jnp.int32),
            pltpu.VMEM((EB,), jnp.int32),
            pltpu.VMEM((EB, D), jnp.float32),
            pltpu.VMEM((EB, D), jnp.float32),
            pltpu.VMEM((EB, D), jnp.float32),
            pltpu.VMEM_SHARED((N, D), jnp.float32),
            pltpu.SemaphoreType.DMA,
            pltpu.SemaphoreType.DMA,
            pltpu.SemaphoreType.DMA,
            pltpu.SemaphoreType.DMA,
            pltpu.SemaphoreType.DMA,
            pltpu.SemaphoreType.DMA,
        ],
    )


def _dinv_block(d0, d1):
    deg = d0 + d1 + 1.0
    return lax.rsqrt(deg)


def _mm_scale_body(x_ref, w_ref, d0_ref, d1_ref, o_ref):
    dinv = _dinv_block(d0_ref[...], d1_ref[...])
    o_ref[...] = jnp.dot(x_ref[...], w_ref[...],
                         preferred_element_type=jnp.float32) * dinv


def _phase0_agg(p0_ref, p1_ref, h2_ref, b_ref, dinv, t_v, st_v):
    i = pl.program_id(1)
    t = dinv * (p0_ref[...] + p1_ref[...] + h2_ref[...]) + b_ref[0:1, :]
    t_v[pl.ds(i * R, R), :] = t

    @pl.when(i == 0)
    def _():
        st_v[...] = jnp.zeros_like(st_v)

    st_v[...] += jnp.concatenate(
        [jnp.sum(t, axis=0, keepdims=True),
         jnp.sum(t * t, axis=0, keepdims=True),
         jnp.zeros((6, D), jnp.float32)], axis=0)


def _bn_relu_from_scratch(g_ref, beta_ref, t_v, st_v):
    i = pl.program_id(1)
    st = st_v[...]
    mean = st[0:1, :] * (1.0 / N)
    var = st[1:2, :] * (1.0 / N) - mean * mean
    xn = ((t_v[pl.ds(i * R, R), :] - mean)
          * (g_ref[0:1, :] * lax.rsqrt(var + EPS)) + beta_ref[0:1, :])
    return jnp.maximum(xn, 0.0)


def _mid_body(p0_ref, p1_ref, h2_ref, d0_ref, d1_ref, b_ref, g_ref, beta_ref,
              w_ref, o_ref, t_v, st_v):
    dinv = _dinv_block(d0_ref[...], d1_ref[...])

    @pl.when(pl.program_id(0) == 0)
    def _p0():
        _phase0_agg(p0_ref, p1_ref, h2_ref, b_ref, dinv, t_v, st_v)

    @pl.when(pl.program_id(0) == 1)
    def _p1():
        xn = _bn_relu_from_scratch(g_ref, beta_ref, t_v, st_v)
        o_ref[...] = jnp.dot(xn, w_ref[...],
                             preferred_element_type=jnp.float32) * dinv


def _fin_body(p0_ref, p1_ref, h2_ref, d0_ref, d1_ref, b_ref, g_ref, beta_ref,
              o_ref, t_v, st_v):
    dinv = _dinv_block(d0_ref[...], d1_ref[...])

    @pl.when(pl.program_id(0) == 0)
    def _p0():
        _phase0_agg(p0_ref, p1_ref, h2_ref, b_ref, dinv, t_v, st_v)

    @pl.when(pl.program_id(0) == 1)
    def _p1():
        o_ref[...] = _bn_relu_from_scratch(g_ref, beta_ref, t_v, st_v)


def _row_spec(width):
    return pl.BlockSpec((R, width), lambda i: (i, 0))


def _full_spec(rows, cols):
    return pl.BlockSpec((rows, cols), lambda i: (0, 0))


_mm_scale = pl.pallas_call(
    _mm_scale_body,
    grid=(G,),
    in_specs=[_row_spec(D), _full_spec(D, D), _row_spec(1), _row_spec(1)],
    out_specs=_row_spec(D),
    out_shape=jax.ShapeDtypeStruct((N, D), jnp.float32),
)

def _row2_spec(width):
    return pl.BlockSpec((R, width), lambda p, i: (i * (1 - p), 0))


def _rowb_spec(width):
    return pl.BlockSpec((R, width), lambda p, i: (i, 0))


def _fullb_spec(rows, cols):
    return pl.BlockSpec((rows, cols), lambda p, i: (0, 0))


_scratch_ts = [pltpu.VMEM((N, D), jnp.float32), pltpu.VMEM((8, D), jnp.float32)]

_mid = pl.pallas_call(
    _mid_body,
    grid=(2, G),
    in_specs=[_row2_spec(D), _row2_spec(D), _row2_spec(D), _rowb_spec(1),
              _rowb_spec(1), _fullb_spec(8, D), _fullb_spec(8, D),
              _fullb_spec(8, D), _fullb_spec(D, D)],
    out_specs=_rowb_spec(D),
    out_shape=jax.ShapeDtypeStruct((N, D), jnp.float32),
    scratch_shapes=_scratch_ts,
)

_fin = pl.pallas_call(
    _fin_body,
    grid=(2, G),
    in_specs=[_row2_spec(D), _row2_spec(D), _row2_spec(D), _rowb_spec(1),
              _rowb_spec(1), _fullb_spec(8, D), _fullb_spec(8, D),
              _fullb_spec(8, D)],
    out_specs=_rowb_spec(D),
    out_shape=jax.ShapeDtypeStruct((N, D), jnp.float32),
    scratch_shapes=_scratch_ts,
)


def _pad8(v):
    return jnp.broadcast_to(v, (8, D))


def kernel(x, edge_index, W1, b1, g1, beta1, W2, b2, g2, beta2):
    src = edge_index[0]
    dst = edge_index[1]
    ones1 = jnp.ones((EB,), jnp.float32)
    zeros1 = jnp.zeros((ZT,), jnp.float32)
    zerosD = jnp.zeros((ZT, D), jnp.float32)

    degf = _deg_call()(dst, ones1, zeros1)
    d0 = degf[:N].reshape(N, 1)
    d1 = degf[N:].reshape(N, 1)

    h2 = _mm_scale(x, W1, d0, d1)
    parts = _scat_call()(h2, src, dst, zerosD)
    h2b = _mid(parts[0], parts[1], h2, d0, d1,
               _pad8(b1), _pad8(g1), _pad8(beta1), W2)
    parts2 = _scat_call()(h2b, src, dst, zerosD)
    return _fin(parts2[0], parts2[1], h2b, d0, d1,
                _pad8(b2), _pad8(g2), _pad8(beta2))
